# baseline (speedup 1.0000x reference)
import functools

import jax
import jax.numpy as jnp
from jax import lax
from jax.experimental import pallas as pl
from jax.experimental.pallas import tpu as pltpu
from jax.experimental.pallas import tpu_sc as plsc


def _sc_lookup(language_id, table):
    b = language_id.shape[0]
    _, d = table.shape
    mesh = plsc.VectorSubcoreMesh(core_axis_name="c", subcore_axis_name="s")

    @functools.partial(
        pl.kernel,
        out_type=jax.ShapeDtypeStruct((b, d), table.dtype),
        mesh=mesh,
        scratch_types=[
            pltpu.VMEM((b,), jnp.int32),
            pltpu.VMEM((b, d), table.dtype),
            pltpu.SemaphoreType.DMA,
        ],
    )
    def lookup(idx_hbm, table_hbm, out_hbm, idx_v, rows_v, sem):
        cid = lax.axis_index("c")
        sid = lax.axis_index("s")

        @pl.when(jnp.logical_and(cid == 0, sid == 0))
        def _():
            pltpu.sync_copy(idx_hbm, idx_v)
            pltpu.async_copy(table_hbm.at[idx_v], rows_v, sem).wait()
            pltpu.sync_copy(rows_v, out_hbm)

    return lookup(language_id, table)


def _tc_broadcast_add(x, lang_emb, tile):
    batch, seq, d = x.shape
    lang_emb = lang_emb.reshape(batch, 1, d)

    def body(x_ref, e_ref, o_ref):
        o_ref[...] = x_ref[...] + e_ref[...]

    return pl.pallas_call(
        body,
        grid=(batch, seq // tile),
        in_specs=[
            pl.BlockSpec((1, tile, d), lambda i, j: (i, j, 0)),
            pl.BlockSpec((1, 1, d), lambda i, j: (i, 0, 0)),
        ],
        out_specs=pl.BlockSpec((1, tile, d), lambda i, j: (i, j, 0)),
        out_shape=jax.ShapeDtypeStruct(x.shape, x.dtype),
        compiler_params=pltpu.CompilerParams(
            dimension_semantics=("parallel", "arbitrary"),
        ),
    )(x, lang_emb)


def kernel(x, language_id, language_embeddings):
    lang_emb = _sc_lookup(language_id.astype(jnp.int32), language_embeddings)
    return _tc_broadcast_add(x, lang_emb, tile=512)


# speedup vs baseline: 1.3825x; 1.3825x over previous
import jax
import jax.numpy as jnp
from jax.experimental import pallas as pl
from jax.experimental.pallas import tpu as pltpu


def kernel(x, language_id, language_embeddings):
    batch, seq, d = x.shape
    tile = 512
    tab3 = language_embeddings[:, None, :]
    lid = language_id.astype(jnp.int32)

    def body(lid_ref, x_ref, e_ref, o_ref):
        o_ref[...] = x_ref[...] + e_ref[...]

    grid_spec = pltpu.PrefetchScalarGridSpec(
        num_scalar_prefetch=1,
        grid=(batch, seq // tile),
        in_specs=[
            pl.BlockSpec((1, tile, d), lambda i, j, lid_ref: (i, j, 0)),
            pl.BlockSpec((1, 1, d), lambda i, j, lid_ref: (lid_ref[i], 0, 0)),
        ],
        out_specs=pl.BlockSpec((1, tile, d), lambda i, j, lid_ref: (i, j, 0)),
    )
    return pl.pallas_call(
        body,
        grid_spec=grid_spec,
        out_shape=jax.ShapeDtypeStruct(x.shape, x.dtype),
        compiler_params=pltpu.CompilerParams(
            dimension_semantics=("arbitrary", "arbitrary"),
        ),
    )(lid, x, tab3)


# speedup vs baseline: 1.5610x; 1.1291x over previous
import jax
import jax.numpy as jnp
from jax.experimental import pallas as pl
from jax.experimental.pallas import tpu as pltpu


def kernel(x, language_id, language_embeddings):
    batch, seq, d = x.shape
    tile = 2048
    tab3 = language_embeddings[:, None, :]
    lid = language_id.astype(jnp.int32)

    def body(lid_ref, x_ref, e_ref, o_ref):
        o_ref[...] = x_ref[...] + e_ref[...]

    grid_spec = pltpu.PrefetchScalarGridSpec(
        num_scalar_prefetch=1,
        grid=(batch, seq // tile),
        in_specs=[
            pl.BlockSpec((1, tile, d), lambda i, j, lid_ref: (i, j, 0)),
            pl.BlockSpec((1, 1, d), lambda i, j, lid_ref: (lid_ref[i], 0, 0)),
        ],
        out_specs=pl.BlockSpec((1, tile, d), lambda i, j, lid_ref: (i, j, 0)),
    )
    return pl.pallas_call(
        body,
        grid_spec=grid_spec,
        out_shape=jax.ShapeDtypeStruct(x.shape, x.dtype),
        compiler_params=pltpu.CompilerParams(
            dimension_semantics=("arbitrary", "arbitrary"),
        ),
    )(lid, x, tab3)


# speedup vs baseline: 1.5616x; 1.0004x over previous
import jax
import jax.numpy as jnp
from jax.experimental import pallas as pl
from jax.experimental.pallas import tpu as pltpu


def kernel(x, language_id, language_embeddings):
    batch, seq, d = x.shape
    tile = 2048
    tab3 = language_embeddings[:, None, :]
    lid = language_id.astype(jnp.int32)

    def body(lid_ref, x_ref, e_ref, o_ref):
        o_ref[...] = x_ref[...] + e_ref[...]

    grid_spec = pltpu.PrefetchScalarGridSpec(
        num_scalar_prefetch=1,
        grid=(batch, seq // tile),
        in_specs=[
            pl.BlockSpec((1, tile, d), lambda i, j, lid_ref: (i, j, 0)),
            pl.BlockSpec((1, 1, d), lambda i, j, lid_ref: (lid_ref[i], 0, 0)),
        ],
        out_specs=pl.BlockSpec((1, tile, d), lambda i, j, lid_ref: (i, j, 0)),
    )
    return pl.pallas_call(
        body,
        grid_spec=grid_spec,
        out_shape=jax.ShapeDtypeStruct(x.shape, x.dtype),
        compiler_params=pltpu.CompilerParams(
            dimension_semantics=("parallel", "parallel"),
        ),
    )(lid, x, tab3)
